# baseline (speedup 1.0000x reference)
import functools

import jax
import jax.numpy as jnp
from jax import lax
from jax.experimental import pallas as pl
from jax.experimental.pallas import tpu as pltpu
from jax.experimental.pallas import tpu_sc as plsc

D = 384; DEPTH = 12; H = 6; DH = 64; FF = 1536; E = 6; NC = 1000; P = 16; G = 14; NTOK = 197
MOE_AT = {8: 0, 10: 1}
BF = jnp.bfloat16
F32 = jnp.float32
I32 = jnp.int32

TPI = 200
NB = 32
NTOT = NB * TPI
IPB = 8
ABR = IPB * TPI

NW = 32
NCH = 5
CHUNK = 40
BLK = 512
NBLK = 18
CAP = NBLK * BLK
SCC = 640


def _bf(x):
    return x.astype(BF)


def _mm(a, b):
    return jax.lax.dot(a.astype(BF), b.astype(BF), preferred_element_type=F32)


def _mm_t(a, b):
    return jax.lax.dot_general(a.astype(BF), b.astype(BF),
                               (((1,), (1,)), ((), ())),
                               preferred_element_type=F32)


def _ln(x, s, b):
    m = x.mean(-1, keepdims=True)
    v = ((x - m) ** 2).mean(-1, keepdims=True)
    return (x - m) / jnp.sqrt(v + 1e-6) * s + b


def _cos_logits(h, re):
    tn = h / (jnp.sqrt((h * h).sum(-1, keepdims=True)) + 1e-6)
    en = re / (jnp.sqrt((re * re).sum(-1, keepdims=True)) + 1e-6)
    return jax.lax.dot_general(tn, en, (((1,), (1,)), ((), ())),
                               preferred_element_type=F32,
                               precision=jax.lax.Precision.HIGHEST)


def _patch_body(p_ref, w_ref, b_ref, pos_ref, o_ref):
    y = _mm(p_ref[0], w_ref[...])
    o_ref[0] = y + b_ref[...] + pos_ref[0]


def _patch_call(p, w_bf, b, pos):
    B = p.shape[0]
    return pl.pallas_call(
        _patch_body,
        grid=(B,),
        in_specs=[pl.BlockSpec((1, G * G, 3 * P * P), lambda i: (i, 0, 0)),
                  pl.BlockSpec((3 * P * P, D), lambda i: (0, 0)),
                  pl.BlockSpec((1, D), lambda i: (0, 0)),
                  pl.BlockSpec((1, G * G, D), lambda i: (0, 0, 0))],
        out_specs=pl.BlockSpec((1, G * G, D), lambda i: (i, 0, 0)),
        out_shape=jax.ShapeDtypeStruct((B, G * G, D), F32),
    )(p, w_bf, b, pos)


def _attn_compute(t2, s_ref, bb_ref, wqkv_ref, bqkv_ref, wp_ref, bp_ref):
    h = _ln(t2, s_ref[...], bb_ref[...])
    qkv = _mm(h, wqkv_ref[...]) + bqkv_ref[...]
    kmask = jnp.where(jax.lax.broadcasted_iota(I32, (1, TPI), 1) >= NTOK,
                      -1e30, 0.0).astype(F32)
    imgs = []
    for im in range(IPB):
        r0 = im * TPI
        heads = []
        for hh in range(H):
            q = qkv[r0:r0 + TPI, hh * DH:(hh + 1) * DH] * 0.125
            k = qkv[r0:r0 + TPI, D + hh * DH:D + (hh + 1) * DH]
            v = qkv[r0:r0 + TPI, 2 * D + hh * DH:2 * D + (hh + 1) * DH]
            sc = _mm_t(q, k) + kmask
            sc = sc - sc.max(-1, keepdims=True)
            ex = jnp.exp(sc)
            heads.append(_mm(ex, v) / ex.sum(-1, keepdims=True))
        imgs.append(jnp.concatenate(heads, axis=-1))
    o = jnp.concatenate(imgs, axis=0)
    return t2 + _mm(o, wp_ref[...]) + bp_ref[...]


def _attn_body(t_ref, s_ref, bb_ref, wqkv_ref, bqkv_ref, wp_ref, bp_ref, o_ref):
    o_ref[...] = _attn_compute(t_ref[...], s_ref, bb_ref, wqkv_ref, bqkv_ref,
                               wp_ref, bp_ref)


def _attn_res_body(t_ref, y_ref, s_ref, bb_ref, wqkv_ref, bqkv_ref, wp_ref,
                   bp_ref, o_ref):
    o_ref[...] = _attn_compute(t_ref[...] + y_ref[...], s_ref, bb_ref,
                               wqkv_ref, bqkv_ref, wp_ref, bp_ref)


_ATTN_WSPECS = [pl.BlockSpec((1, D), lambda i: (0, 0)),
                pl.BlockSpec((1, D), lambda i: (0, 0)),
                pl.BlockSpec((D, 3 * D), lambda i: (0, 0)),
                pl.BlockSpec((1, 3 * D), lambda i: (0, 0)),
                pl.BlockSpec((D, D), lambda i: (0, 0)),
                pl.BlockSpec((1, D), lambda i: (0, 0))]


def _attn_call(tf, s, b, wqkv, bqkv, wp, bp):
    return pl.pallas_call(
        _attn_body,
        grid=(NTOT // ABR,),
        in_specs=[pl.BlockSpec((ABR, D), lambda i: (i, 0))] + _ATTN_WSPECS,
        out_specs=pl.BlockSpec((ABR, D), lambda i: (i, 0)),
        out_shape=jax.ShapeDtypeStruct((NTOT, D), F32),
    )(tf, s, b, wqkv, bqkv, wp, bp)


def _attn_res_call(tf, y, s, b, wqkv, bqkv, wp, bp):
    return pl.pallas_call(
        _attn_res_body,
        grid=(NTOT // ABR,),
        in_specs=[pl.BlockSpec((ABR, D), lambda i: (i, 0)),
                  pl.BlockSpec((ABR, D), lambda i: (i, 0))] + _ATTN_WSPECS,
        out_specs=pl.BlockSpec((ABR, D), lambda i: (i, 0)),
        out_shape=jax.ShapeDtypeStruct((NTOT, D), F32),
    )(tf, y, s, b, wqkv, bqkv, wp, bp)


def _ffn_body(t_ref, s_ref, b_ref, w1_ref, b1_ref, w2_ref, b2_ref, o_ref):
    t = t_ref[...]
    h = _ln(t, s_ref[...], b_ref[...])
    a = jax.nn.gelu(_mm(h, w1_ref[...]) + b1_ref[...])
    o_ref[...] = t + _mm(a, w2_ref[...]) + b2_ref[...]


def _ffn_call(tf, s, b, w1, b1, w2, b2):
    return pl.pallas_call(
        _ffn_body,
        grid=(NTOT // ABR,),
        in_specs=[pl.BlockSpec((ABR, D), lambda i: (i, 0)),
                  pl.BlockSpec((1, D), lambda i: (0, 0)),
                  pl.BlockSpec((1, D), lambda i: (0, 0)),
                  pl.BlockSpec((D, FF), lambda i: (0, 0)),
                  pl.BlockSpec((1, FF), lambda i: (0, 0)),
                  pl.BlockSpec((FF, D), lambda i: (0, 0)),
                  pl.BlockSpec((1, D), lambda i: (0, 0))],
        out_specs=pl.BlockSpec((ABR, D), lambda i: (i, 0)),
        out_shape=jax.ShapeDtypeStruct((NTOT, D), F32),
    )(tf, s, b, w1, b1, w2, b2)


def _hlog_body(t_ref, s_ref, b_ref, re_ref, h_ref, lg_ref):
    h = _ln(t_ref[...], s_ref[...], b_ref[...])
    h_ref[...] = h
    lg_ref[...] = _cos_logits(h, re_ref[...])


def _hlog_call(tf, s, b, re):
    return pl.pallas_call(
        _hlog_body,
        grid=(NTOT // ABR,),
        in_specs=[pl.BlockSpec((ABR, D), lambda i: (i, 0)),
                  pl.BlockSpec((1, D), lambda i: (0, 0)),
                  pl.BlockSpec((1, D), lambda i: (0, 0)),
                  pl.BlockSpec((E, D), lambda i: (0, 0))],
        out_specs=[pl.BlockSpec((ABR, D), lambda i: (i, 0)),
                   pl.BlockSpec((ABR, E), lambda i: (i, 0))],
        out_shape=[jax.ShapeDtypeStruct((NTOT, D), F32),
                   jax.ShapeDtypeStruct((NTOT, E), F32)],
    )(tf, s, b, re)


def _route_body(lg_ref, dst_ref, bexp_ref):
    lg = lg_ref[...]
    amax = jnp.argmax(lg, axis=-1)
    oh = (jax.lax.broadcasted_iota(I32, (NTOT, E), 1)
          == amax[:, None]).astype(F32)
    tri = (jax.lax.broadcasted_iota(I32, (SCC, SCC), 0)
           > jax.lax.broadcasted_iota(I32, (SCC, SCC), 1)).astype(F32)
    offs = jnp.zeros((1, E), F32)
    parts = []
    for c in range(NTOT // SCC):
        ohc = oh[c * SCC:(c + 1) * SCC]
        excl = jax.lax.dot(tri, ohc, preferred_element_type=F32,
                           precision=jax.lax.Precision.HIGHEST)
        parts.append(excl + offs)
        offs = offs + ohc.sum(0, keepdims=True)
    rank = jnp.concatenate(parts, axis=0)
    counts = offs
    padded = jnp.ceil(counts / BLK) * BLK
    su = (jax.lax.broadcasted_iota(I32, (E, E), 0)
          < jax.lax.broadcasted_iota(I32, (E, E), 1)).astype(F32)
    base = jax.lax.dot(padded, su, preferred_element_type=F32,
                       precision=jax.lax.Precision.HIGHEST)
    dst = ((rank + base) * oh).sum(-1, keepdims=True)
    dst_ref[...] = dst.astype(I32)
    jb = jax.lax.broadcasted_iota(I32, (NBLK, 1), 0).astype(F32) * BLK
    active = (jb >= base) & (jb < base + padded)
    eidx = jax.lax.broadcasted_iota(I32, (NBLK, E), 1).astype(F32)
    bexp_ref[...] = jnp.where(active, eidx, 0.).sum(-1, keepdims=True).astype(I32)


def _route_call(lg):
    return pl.pallas_call(
        _route_body,
        in_specs=[pl.BlockSpec((NTOT, E), lambda: (0, 0))],
        out_specs=[pl.BlockSpec((NTOT, 1), lambda: (0, 0)),
                   pl.BlockSpec((NBLK, 1), lambda: (0, 0))],
        out_shape=[jax.ShapeDtypeStruct((NTOT, 1), I32),
                   jax.ShapeDtypeStruct((NBLK, 1), I32)],
    )(lg)


def _disp_body(h_hbm, idx_hbm, out_hbm, idx_v, rows_v, sem):
    wid = lax.axis_index("s") * 2 + lax.axis_index("c")
    pltpu.sync_copy(idx_hbm.at[wid], idx_v)
    for c in range(NCH):
        base = wid * (NCH * CHUNK) + c * CHUNK
        pltpu.sync_copy(h_hbm.at[pl.ds(base, CHUNK)], rows_v)
        pltpu.async_copy(rows_v, out_hbm.at[idx_v.at[c]], sem).wait()


def _comb_body(y_hbm, idx_hbm, out_hbm, idx_v, rows_v, sem):
    wid = lax.axis_index("s") * 2 + lax.axis_index("c")
    pltpu.sync_copy(idx_hbm.at[wid], idx_v)
    for c in range(NCH):
        base = wid * (NCH * CHUNK) + c * CHUNK
        pltpu.async_copy(y_hbm.at[idx_v.at[c]], rows_v, sem).wait()
        pltpu.sync_copy(rows_v, out_hbm.at[pl.ds(base, CHUNK)])


@functools.cache
def _sc_build():
    mesh = plsc.VectorSubcoreMesh(core_axis_name="c", subcore_axis_name="s",
                                  num_cores=2, num_subcores=16)
    scratch = [pltpu.VMEM((NCH, CHUNK), I32),
               pltpu.VMEM((CHUNK, D), F32),
               pltpu.SemaphoreType.DMA]
    disp = pl.kernel(_disp_body,
                     out_type=jax.ShapeDtypeStruct((CAP, D), F32),
                     mesh=mesh, scratch_types=scratch)
    comb = pl.kernel(_comb_body,
                     out_type=jax.ShapeDtypeStruct((NTOT, D), F32),
                     mesh=mesh, scratch_types=scratch)
    return disp, comb


def _sc_dispatch(h, idx):
    return _sc_build()[0](h, idx)


def _sc_combine(y, idx):
    return _sc_build()[1](y, idx)


def _gffn_body(bexp_ref, h_ref, re_ref, w1_ref, b1_ref, w2_ref, b2_ref, o_ref):
    e = bexp_ref[pl.program_id(0)]
    h = h_ref[...]
    lg = _cos_logits(h, re_ref[...])
    mx = lg.max(-1, keepdims=True)
    ex = jnp.exp(lg - mx)
    probs = ex / ex.sum(-1, keepdims=True)
    onehot_e = (jax.lax.broadcasted_iota(I32, (BLK, E), 1) == e).astype(F32)
    gate = (probs * onehot_e).sum(-1, keepdims=True)
    a = jax.nn.gelu(_mm(h, w1_ref[0]) + b1_ref[0])
    y = _mm(a, w2_ref[0]) + b2_ref[0]
    o_ref[...] = y * gate


def _gffn_call(bexp, sorted_h, re, w1, b1, w2, b2):
    grid_spec = pltpu.PrefetchScalarGridSpec(
        num_scalar_prefetch=1,
        grid=(NBLK,),
        in_specs=[pl.BlockSpec((BLK, D), lambda i, s: (i, 0)),
                  pl.BlockSpec((E, D), lambda i, s: (0, 0)),
                  pl.BlockSpec((1, D, FF), lambda i, s: (s[i], 0, 0)),
                  pl.BlockSpec((1, 1, FF), lambda i, s: (s[i], 0, 0)),
                  pl.BlockSpec((1, FF, D), lambda i, s: (s[i], 0, 0)),
                  pl.BlockSpec((1, 1, D), lambda i, s: (s[i], 0, 0))],
        out_specs=pl.BlockSpec((BLK, D), lambda i, s: (i, 0)),
    )
    return pl.pallas_call(
        _gffn_body,
        grid_spec=grid_spec,
        out_shape=jax.ShapeDtypeStruct((CAP, D), F32),
    )(bexp, sorted_h, re, w1, b1, w2, b2)


def _moe_sparse(tf, s, b, re, w1, b1, w2, b2):
    h, lg = _hlog_call(tf, s, b, re)
    dst, bexp = _route_call(lg)
    idx = dst.reshape(NW, NCH, CHUNK)
    sorted_h = _sc_dispatch(h, idx)
    sorted_y = _gffn_call(bexp.reshape(NBLK), sorted_h, re,
                          _bf(w1), b1.reshape(E, 1, FF),
                          _bf(w2), b2.reshape(E, 1, D))
    return _sc_combine(sorted_y, idx)


def _head_body(t_ref, s_ref, b_ref, w_ref, bh_ref, o_ref):
    h = _ln(t_ref[...], s_ref[...], b_ref[...])
    o_ref[...] = _mm(h, w_ref[...]) + bh_ref[...]


def _head_call(tc, s, b, w, bh):
    B = tc.shape[0]
    return pl.pallas_call(
        _head_body,
        in_specs=[pl.BlockSpec((B, D), lambda: (0, 0)),
                  pl.BlockSpec((1, D), lambda: (0, 0)),
                  pl.BlockSpec((1, D), lambda: (0, 0)),
                  pl.BlockSpec((D, NC), lambda: (0, 0)),
                  pl.BlockSpec((1, NC), lambda: (0, 0))],
        out_specs=pl.BlockSpec((B, NC), lambda: (0, 0)),
        out_shape=jax.ShapeDtypeStruct((B, NC), F32),
    )(tc, s, b, w, bh)


def kernel(x, patch_w, patch_b, cls_tok, pos_emb, ln1_s, ln1_b, qkv_w, qkv_b,
           proj_w, proj_b, ln2_s, ln2_b, fc1_w, fc1_b, fc2_w, fc2_b,
           moe_w1, moe_b1, moe_w2, moe_b2, router_e, lnf_s, lnf_b,
           head_w, head_b):
    B = x.shape[0]
    p = x.reshape(B, 3, G, P, G, P).transpose(0, 2, 4, 3, 5, 1).reshape(B, G * G, 3 * P * P)
    t0 = _patch_call(p, _bf(patch_w), patch_b.reshape(1, D), pos_emb[:, 1:])
    cls_row = cls_tok + pos_emb[:, 0:1]
    t3 = jnp.concatenate([jnp.broadcast_to(cls_row, (B, 1, D)), t0,
                          jnp.zeros((B, TPI - NTOK, D), F32)], axis=1)
    tf = t3.reshape(NTOT, D)
    pending = None
    for i in range(DEPTH):
        args = (ln1_s[i].reshape(1, D), ln1_b[i].reshape(1, D),
                _bf(qkv_w[i]), qkv_b[i].reshape(1, 3 * D),
                _bf(proj_w[i]), proj_b[i].reshape(1, D))
        if pending is None:
            tf = _attn_call(tf, *args)
        else:
            tf = _attn_res_call(tf, pending, *args)
            pending = None
        if i in MOE_AT:
            j = MOE_AT[i]
            pending = _moe_sparse(tf, ln2_s[i].reshape(1, D),
                                  ln2_b[i].reshape(1, D),
                                  router_e[j], moe_w1[j], moe_b1[j],
                                  moe_w2[j], moe_b2[j])
        else:
            tf = _ffn_call(tf, ln2_s[i].reshape(1, D), ln2_b[i].reshape(1, D),
                           _bf(fc1_w[i]), fc1_b[i].reshape(1, FF),
                           _bf(fc2_w[i]), fc2_b[i].reshape(1, D))
    tc = tf.reshape(B, TPI, D)[:, 0]
    return _head_call(tc, lnf_s.reshape(1, D), lnf_b.reshape(1, D),
                      _bf(head_w), head_b.reshape(1, NC))


# speedup vs baseline: 1.0259x; 1.0259x over previous
import functools

import jax
import jax.numpy as jnp
from jax import lax
from jax.experimental import pallas as pl
from jax.experimental.pallas import tpu as pltpu
from jax.experimental.pallas import tpu_sc as plsc

D = 384; DEPTH = 12; H = 6; DH = 64; FF = 1536; E = 6; NC = 1000; P = 16; G = 14; NTOK = 197
MOE_AT = {8: 0, 10: 1}
BF = jnp.bfloat16
F32 = jnp.float32
I32 = jnp.int32

TPI = 200
NB = 32
NTOT = NB * TPI
IPB = 8
ABR = IPB * TPI

NW = 32
NCH = 5
CHUNK = 40
BLK = 512
NBLK = 18
CAP = NBLK * BLK
SCC = 640


def _bf(x):
    return x.astype(BF)


def _mm(a, b):
    return jax.lax.dot(a.astype(BF), b.astype(BF), preferred_element_type=F32)


def _mm_t(a, b):
    return jax.lax.dot_general(a.astype(BF), b.astype(BF),
                               (((1,), (1,)), ((), ())),
                               preferred_element_type=F32)


def _ln(x, s, b):
    m = x.mean(-1, keepdims=True)
    v = ((x - m) ** 2).mean(-1, keepdims=True)
    return (x - m) / jnp.sqrt(v + 1e-6) * s + b


def _cos_logits(h, re):
    tn = h / (jnp.sqrt((h * h).sum(-1, keepdims=True)) + 1e-6)
    en = re / (jnp.sqrt((re * re).sum(-1, keepdims=True)) + 1e-6)
    return jax.lax.dot_general(tn, en, (((1,), (1,)), ((), ())),
                               preferred_element_type=F32,
                               precision=jax.lax.Precision.HIGHEST)


def _patch_body(p_ref, w_ref, b_ref, pos_ref, o_ref):
    y = _mm(p_ref[0], w_ref[...])
    o_ref[0] = y + b_ref[...] + pos_ref[0]


def _patch_call(p, w_bf, b, pos):
    B = p.shape[0]
    return pl.pallas_call(
        _patch_body,
        grid=(B,),
        in_specs=[pl.BlockSpec((1, G * G, 3 * P * P), lambda i: (i, 0, 0)),
                  pl.BlockSpec((3 * P * P, D), lambda i: (0, 0)),
                  pl.BlockSpec((1, D), lambda i: (0, 0)),
                  pl.BlockSpec((1, G * G, D), lambda i: (0, 0, 0))],
        out_specs=pl.BlockSpec((1, G * G, D), lambda i: (i, 0, 0)),
        out_shape=jax.ShapeDtypeStruct((B, G * G, D), F32),
    )(p, w_bf, b, pos)


def _attn_compute(t2, s_ref, bb_ref, wqkv_ref, bqkv_ref, wp_ref, bp_ref):
    h = _ln(t2, s_ref[...], bb_ref[...])
    qkv = _mm(h, wqkv_ref[...]) + bqkv_ref[...]
    kmask = jnp.where(jax.lax.broadcasted_iota(I32, (1, TPI), 1) >= NTOK,
                      -1e30, 0.0).astype(F32)
    imgs = []
    for im in range(IPB):
        r0 = im * TPI
        heads = []
        for hh in range(H):
            q = qkv[r0:r0 + TPI, hh * DH:(hh + 1) * DH]
            k = qkv[r0:r0 + TPI, D + hh * DH:D + (hh + 1) * DH]
            v = qkv[r0:r0 + TPI, 2 * D + hh * DH:2 * D + (hh + 1) * DH]
            sc = _mm_t(q, k) * 0.125 + kmask
            sc = sc - sc.max(-1, keepdims=True)
            ex = jnp.exp(sc)
            a = ex / ex.sum(-1, keepdims=True)
            heads.append(_mm(a, v))
        imgs.append(jnp.concatenate(heads, axis=-1))
    o = jnp.concatenate(imgs, axis=0)
    return t2 + _mm(o, wp_ref[...]) + bp_ref[...]


def _attn_body(t_ref, s_ref, bb_ref, wqkv_ref, bqkv_ref, wp_ref, bp_ref, o_ref):
    o_ref[...] = _attn_compute(t_ref[...], s_ref, bb_ref, wqkv_ref, bqkv_ref,
                               wp_ref, bp_ref)


def _attn_res_body(t_ref, y_ref, s_ref, bb_ref, wqkv_ref, bqkv_ref, wp_ref,
                   bp_ref, o_ref):
    o_ref[...] = _attn_compute(t_ref[...] + y_ref[...], s_ref, bb_ref,
                               wqkv_ref, bqkv_ref, wp_ref, bp_ref)


_ATTN_WSPECS = [pl.BlockSpec((1, D), lambda i: (0, 0)),
                pl.BlockSpec((1, D), lambda i: (0, 0)),
                pl.BlockSpec((D, 3 * D), lambda i: (0, 0)),
                pl.BlockSpec((1, 3 * D), lambda i: (0, 0)),
                pl.BlockSpec((D, D), lambda i: (0, 0)),
                pl.BlockSpec((1, D), lambda i: (0, 0))]


def _attn_call(tf, s, b, wqkv, bqkv, wp, bp):
    return pl.pallas_call(
        _attn_body,
        grid=(NTOT // ABR,),
        in_specs=[pl.BlockSpec((ABR, D), lambda i: (i, 0))] + _ATTN_WSPECS,
        out_specs=pl.BlockSpec((ABR, D), lambda i: (i, 0)),
        out_shape=jax.ShapeDtypeStruct((NTOT, D), F32),
    )(tf, s, b, wqkv, bqkv, wp, bp)


def _attn_res_call(tf, y, s, b, wqkv, bqkv, wp, bp):
    return pl.pallas_call(
        _attn_res_body,
        grid=(NTOT // ABR,),
        in_specs=[pl.BlockSpec((ABR, D), lambda i: (i, 0)),
                  pl.BlockSpec((ABR, D), lambda i: (i, 0))] + _ATTN_WSPECS,
        out_specs=pl.BlockSpec((ABR, D), lambda i: (i, 0)),
        out_shape=jax.ShapeDtypeStruct((NTOT, D), F32),
    )(tf, y, s, b, wqkv, bqkv, wp, bp)


def _ffn_body(t_ref, s_ref, b_ref, w1_ref, b1_ref, w2_ref, b2_ref, o_ref):
    t = t_ref[...]
    h = _ln(t, s_ref[...], b_ref[...])
    a = jax.nn.gelu(_mm(h, w1_ref[...]) + b1_ref[...])
    o_ref[...] = t + _mm(a, w2_ref[...]) + b2_ref[...]


def _ffn_call(tf, s, b, w1, b1, w2, b2):
    return pl.pallas_call(
        _ffn_body,
        grid=(NTOT // ABR,),
        in_specs=[pl.BlockSpec((ABR, D), lambda i: (i, 0)),
                  pl.BlockSpec((1, D), lambda i: (0, 0)),
                  pl.BlockSpec((1, D), lambda i: (0, 0)),
                  pl.BlockSpec((D, FF), lambda i: (0, 0)),
                  pl.BlockSpec((1, FF), lambda i: (0, 0)),
                  pl.BlockSpec((FF, D), lambda i: (0, 0)),
                  pl.BlockSpec((1, D), lambda i: (0, 0))],
        out_specs=pl.BlockSpec((ABR, D), lambda i: (i, 0)),
        out_shape=jax.ShapeDtypeStruct((NTOT, D), F32),
    )(tf, s, b, w1, b1, w2, b2)


def _hlog_body(t_ref, s_ref, b_ref, re_ref, h_ref, lg_ref):
    h = _ln(t_ref[...], s_ref[...], b_ref[...])
    h_ref[...] = h
    lg_ref[...] = _cos_logits(h, re_ref[...])


def _hlog_call(tf, s, b, re):
    return pl.pallas_call(
        _hlog_body,
        grid=(NTOT // ABR,),
        in_specs=[pl.BlockSpec((ABR, D), lambda i: (i, 0)),
                  pl.BlockSpec((1, D), lambda i: (0, 0)),
                  pl.BlockSpec((1, D), lambda i: (0, 0)),
                  pl.BlockSpec((E, D), lambda i: (0, 0))],
        out_specs=[pl.BlockSpec((ABR, D), lambda i: (i, 0)),
                   pl.BlockSpec((ABR, E), lambda i: (i, 0))],
        out_shape=[jax.ShapeDtypeStruct((NTOT, D), F32),
                   jax.ShapeDtypeStruct((NTOT, E), F32)],
    )(tf, s, b, re)


def _route_body(lg_ref, dst_ref, bexp_ref):
    lg = lg_ref[...]
    amax = jnp.argmax(lg, axis=-1)
    oh = (jax.lax.broadcasted_iota(I32, (NTOT, E), 1)
          == amax[:, None]).astype(F32)
    tri = (jax.lax.broadcasted_iota(I32, (SCC, SCC), 0)
           > jax.lax.broadcasted_iota(I32, (SCC, SCC), 1)).astype(F32)
    offs = jnp.zeros((1, E), F32)
    parts = []
    for c in range(NTOT // SCC):
        ohc = oh[c * SCC:(c + 1) * SCC]
        excl = jax.lax.dot(tri, ohc, preferred_element_type=F32,
                           precision=jax.lax.Precision.HIGHEST)
        parts.append(excl + offs)
        offs = offs + ohc.sum(0, keepdims=True)
    rank = jnp.concatenate(parts, axis=0)
    counts = offs
    padded = jnp.ceil(counts / BLK) * BLK
    su = (jax.lax.broadcasted_iota(I32, (E, E), 0)
          < jax.lax.broadcasted_iota(I32, (E, E), 1)).astype(F32)
    base = jax.lax.dot(padded, su, preferred_element_type=F32,
                       precision=jax.lax.Precision.HIGHEST)
    dst = ((rank + base) * oh).sum(-1, keepdims=True)
    dst_ref[...] = dst.astype(I32).reshape(NW * NCH, CHUNK)
    jb = jax.lax.broadcasted_iota(I32, (NBLK, 1), 0).astype(F32) * BLK
    active = (jb >= base) & (jb < base + padded)
    eidx = jax.lax.broadcasted_iota(I32, (NBLK, E), 1).astype(F32)
    bexp_ref[...] = jnp.where(active, eidx, 0.).sum(-1, keepdims=True).astype(I32)


def _route_call(lg):
    return pl.pallas_call(
        _route_body,
        in_specs=[pl.BlockSpec((NTOT, E), lambda: (0, 0))],
        out_specs=[pl.BlockSpec((NW * NCH, CHUNK), lambda: (0, 0)),
                   pl.BlockSpec((NBLK, 1), lambda: (0, 0))],
        out_shape=[jax.ShapeDtypeStruct((NW * NCH, CHUNK), I32),
                   jax.ShapeDtypeStruct((NBLK, 1), I32)],
    )(lg)


def _disp_body(h_hbm, idx_hbm, out_hbm, idx_v, rows_v, sem):
    wid = lax.axis_index("s") * 2 + lax.axis_index("c")
    pltpu.sync_copy(idx_hbm.at[wid], idx_v)
    for c in range(NCH):
        base = wid * (NCH * CHUNK) + c * CHUNK
        pltpu.sync_copy(h_hbm.at[pl.ds(base, CHUNK)], rows_v)
        pltpu.async_copy(rows_v, out_hbm.at[idx_v.at[c]], sem).wait()


def _comb_body(y_hbm, idx_hbm, out_hbm, idx_v, rows_v, sem):
    wid = lax.axis_index("s") * 2 + lax.axis_index("c")
    pltpu.sync_copy(idx_hbm.at[wid], idx_v)
    for c in range(NCH):
        base = wid * (NCH * CHUNK) + c * CHUNK
        pltpu.async_copy(y_hbm.at[idx_v.at[c]], rows_v, sem).wait()
        pltpu.sync_copy(rows_v, out_hbm.at[pl.ds(base, CHUNK)])


@functools.cache
def _sc_build():
    mesh = plsc.VectorSubcoreMesh(core_axis_name="c", subcore_axis_name="s",
                                  num_cores=2, num_subcores=16)
    scratch = [pltpu.VMEM((NCH, CHUNK), I32),
               pltpu.VMEM((CHUNK, D), F32),
               pltpu.SemaphoreType.DMA]
    disp = pl.kernel(_disp_body,
                     out_type=jax.ShapeDtypeStruct((CAP, D), F32),
                     mesh=mesh, scratch_types=scratch)
    comb = pl.kernel(_comb_body,
                     out_type=jax.ShapeDtypeStruct((NTOT, D), F32),
                     mesh=mesh, scratch_types=scratch)
    return disp, comb


def _sc_dispatch(h, idx):
    return _sc_build()[0](h, idx)


def _sc_combine(y, idx):
    return _sc_build()[1](y, idx)


def _gffn_body(bexp_ref, h_ref, re_ref, w1_ref, b1_ref, w2_ref, b2_ref, o_ref):
    e = bexp_ref[pl.program_id(0)]
    h = h_ref[...]
    lg = _cos_logits(h, re_ref[...])
    mx = lg.max(-1, keepdims=True)
    ex = jnp.exp(lg - mx)
    probs = ex / ex.sum(-1, keepdims=True)
    onehot_e = (jax.lax.broadcasted_iota(I32, (BLK, E), 1) == e).astype(F32)
    gate = (probs * onehot_e).sum(-1, keepdims=True)
    a = jax.nn.gelu(_mm(h, w1_ref[0]) + b1_ref[0])
    y = _mm(a, w2_ref[0]) + b2_ref[0]
    o_ref[...] = y * gate


def _gffn_call(bexp, sorted_h, re, w1, b1, w2, b2):
    grid_spec = pltpu.PrefetchScalarGridSpec(
        num_scalar_prefetch=1,
        grid=(NBLK,),
        in_specs=[pl.BlockSpec((BLK, D), lambda i, s: (i, 0)),
                  pl.BlockSpec((E, D), lambda i, s: (0, 0)),
                  pl.BlockSpec((1, D, FF), lambda i, s: (s[i], 0, 0)),
                  pl.BlockSpec((1, 1, FF), lambda i, s: (s[i], 0, 0)),
                  pl.BlockSpec((1, FF, D), lambda i, s: (s[i], 0, 0)),
                  pl.BlockSpec((1, 1, D), lambda i, s: (s[i], 0, 0))],
        out_specs=pl.BlockSpec((BLK, D), lambda i, s: (i, 0)),
    )
    return pl.pallas_call(
        _gffn_body,
        grid_spec=grid_spec,
        out_shape=jax.ShapeDtypeStruct((CAP, D), F32),
    )(bexp, sorted_h, re, w1, b1, w2, b2)


def _moe_sparse(tf, s, b, re, w1, b1, w2, b2):
    h, lg = _hlog_call(tf, s, b, re)
    dst, bexp = _route_call(lg)
    idx = dst.reshape(NW, NCH, CHUNK)
    sorted_h = _sc_dispatch(h, idx)
    sorted_y = _gffn_call(bexp.reshape(NBLK), sorted_h, re,
                          _bf(w1), b1.reshape(E, 1, FF),
                          _bf(w2), b2.reshape(E, 1, D))
    return _sc_combine(sorted_y, idx)


def _head_body(t_ref, s_ref, b_ref, w_ref, bh_ref, o_ref):
    h = _ln(t_ref[...], s_ref[...], b_ref[...])
    o_ref[...] = _mm(h, w_ref[...]) + bh_ref[...]


def _head_call(tc, s, b, w, bh):
    B = tc.shape[0]
    return pl.pallas_call(
        _head_body,
        in_specs=[pl.BlockSpec((B, D), lambda: (0, 0)),
                  pl.BlockSpec((1, D), lambda: (0, 0)),
                  pl.BlockSpec((1, D), lambda: (0, 0)),
                  pl.BlockSpec((D, NC), lambda: (0, 0)),
                  pl.BlockSpec((1, NC), lambda: (0, 0))],
        out_specs=pl.BlockSpec((B, NC), lambda: (0, 0)),
        out_shape=jax.ShapeDtypeStruct((B, NC), F32),
    )(tc, s, b, w, bh)


def kernel(x, patch_w, patch_b, cls_tok, pos_emb, ln1_s, ln1_b, qkv_w, qkv_b,
           proj_w, proj_b, ln2_s, ln2_b, fc1_w, fc1_b, fc2_w, fc2_b,
           moe_w1, moe_b1, moe_w2, moe_b2, router_e, lnf_s, lnf_b,
           head_w, head_b):
    B = x.shape[0]
    p = x.reshape(B, 3, G, P, G, P).transpose(0, 2, 4, 3, 5, 1).reshape(B, G * G, 3 * P * P)
    t0 = _patch_call(p, _bf(patch_w), patch_b.reshape(1, D), pos_emb[:, 1:])
    cls_row = cls_tok + pos_emb[:, 0:1]
    t3 = jnp.concatenate([jnp.broadcast_to(cls_row, (B, 1, D)), t0,
                          jnp.zeros((B, TPI - NTOK, D), F32)], axis=1)
    tf = t3.reshape(NTOT, D)
    pending = None
    for i in range(DEPTH):
        args = (ln1_s[i].reshape(1, D), ln1_b[i].reshape(1, D),
                _bf(qkv_w[i]), qkv_b[i].reshape(1, 3 * D),
                _bf(proj_w[i]), proj_b[i].reshape(1, D))
        if pending is None:
            tf = _attn_call(tf, *args)
        else:
            tf = _attn_res_call(tf, pending, *args)
            pending = None
        if i in MOE_AT:
            j = MOE_AT[i]
            pending = _moe_sparse(tf, ln2_s[i].reshape(1, D),
                                  ln2_b[i].reshape(1, D),
                                  router_e[j], moe_w1[j], moe_b1[j],
                                  moe_w2[j], moe_b2[j])
        else:
            tf = _ffn_call(tf, ln2_s[i].reshape(1, D), ln2_b[i].reshape(1, D),
                           _bf(fc1_w[i]), fc1_b[i].reshape(1, FF),
                           _bf(fc2_w[i]), fc2_b[i].reshape(1, D))
    tc = tf.reshape(B, TPI, D)[:, 0]
    return _head_call(tc, lnf_s.reshape(1, D), lnf_b.reshape(1, D),
                      _bf(head_w), head_b.reshape(1, NC))


# speedup vs baseline: 1.0261x; 1.0003x over previous
import functools

import jax
import jax.numpy as jnp
from jax import lax
from jax.experimental import pallas as pl
from jax.experimental.pallas import tpu as pltpu
from jax.experimental.pallas import tpu_sc as plsc

D = 384; DEPTH = 12; H = 6; DH = 64; FF = 1536; E = 6; NC = 1000; P = 16; G = 14; NTOK = 197
MOE_AT = {8: 0, 10: 1}
BF = jnp.bfloat16
F32 = jnp.float32
I32 = jnp.int32

TPI = 200
NB = 32
NTOT = NB * TPI
IPB = 8
ABR = IPB * TPI

NW = 32
NCH = 5
CHUNK = 40
BLK = 512
NBLK = 18
CAP = NBLK * BLK
SCC = 640


def _bf(x):
    return x.astype(BF)


def _mm(a, b):
    return jax.lax.dot(a.astype(BF), b.astype(BF), preferred_element_type=F32)


def _mm_t(a, b):
    return jax.lax.dot_general(a.astype(BF), b.astype(BF),
                               (((1,), (1,)), ((), ())),
                               preferred_element_type=F32)


def _ln(x, s, b):
    m = x.mean(-1, keepdims=True)
    v = ((x - m) ** 2).mean(-1, keepdims=True)
    return (x - m) / jnp.sqrt(v + 1e-6) * s + b


def _cos_logits(h, re):
    tn = h / (jnp.sqrt((h * h).sum(-1, keepdims=True)) + 1e-6)
    en = re / (jnp.sqrt((re * re).sum(-1, keepdims=True)) + 1e-6)
    return jax.lax.dot_general(tn, en, (((1,), (1,)), ((), ())),
                               preferred_element_type=F32,
                               precision=jax.lax.Precision.HIGHEST)


def _patch_body(p_ref, w_ref, b_ref, pos_ref, o_ref):
    y = _mm(p_ref[0], w_ref[...])
    o_ref[0] = y + b_ref[...] + pos_ref[0]


def _patch_call(p, w_bf, b, pos):
    B = p.shape[0]
    return pl.pallas_call(
        _patch_body,
        grid=(B,),
        in_specs=[pl.BlockSpec((1, G * G, 3 * P * P), lambda i: (i, 0, 0)),
                  pl.BlockSpec((3 * P * P, D), lambda i: (0, 0)),
                  pl.BlockSpec((1, D), lambda i: (0, 0)),
                  pl.BlockSpec((1, G * G, D), lambda i: (0, 0, 0))],
        out_specs=pl.BlockSpec((1, G * G, D), lambda i: (i, 0, 0)),
        out_shape=jax.ShapeDtypeStruct((B, G * G, D), F32),
    )(p, w_bf, b, pos)


def _attn_compute(t2, s_ref, bb_ref, wqkv_ref, bqkv_ref, wp_ref, bp_ref):
    h = _ln(t2, s_ref[...], bb_ref[...])
    qkv = _mm(h, wqkv_ref[...]) + bqkv_ref[...]
    kmask = jnp.where(jax.lax.broadcasted_iota(I32, (1, TPI), 1) >= NTOK,
                      -1e30, 0.0).astype(F32)
    imgs = []
    for im in range(IPB):
        r0 = im * TPI
        heads = []
        for hh in range(H):
            q = qkv[r0:r0 + TPI, hh * DH:(hh + 1) * DH]
            k = qkv[r0:r0 + TPI, D + hh * DH:D + (hh + 1) * DH]
            v = qkv[r0:r0 + TPI, 2 * D + hh * DH:2 * D + (hh + 1) * DH]
            sc = _mm_t(q, k) * 0.125 + kmask
            sc = sc - sc.max(-1, keepdims=True)
            ex = jnp.exp(sc)
            a = ex / ex.sum(-1, keepdims=True)
            heads.append(_mm(a, v))
        imgs.append(jnp.concatenate(heads, axis=-1))
    o = jnp.concatenate(imgs, axis=0)
    return t2 + _mm(o, wp_ref[...]) + bp_ref[...]


def _attn_body(t_ref, s_ref, bb_ref, wqkv_ref, bqkv_ref, wp_ref, bp_ref, o_ref):
    o_ref[...] = _attn_compute(t_ref[...], s_ref, bb_ref, wqkv_ref, bqkv_ref,
                               wp_ref, bp_ref)


def _attn_res_body(t_ref, y_ref, s_ref, bb_ref, wqkv_ref, bqkv_ref, wp_ref,
                   bp_ref, o_ref):
    o_ref[...] = _attn_compute(t_ref[...] + y_ref[...], s_ref, bb_ref,
                               wqkv_ref, bqkv_ref, wp_ref, bp_ref)


_ATTN_WSPECS = [pl.BlockSpec((1, D), lambda i: (0, 0)),
                pl.BlockSpec((1, D), lambda i: (0, 0)),
                pl.BlockSpec((D, 3 * D), lambda i: (0, 0)),
                pl.BlockSpec((1, 3 * D), lambda i: (0, 0)),
                pl.BlockSpec((D, D), lambda i: (0, 0)),
                pl.BlockSpec((1, D), lambda i: (0, 0))]


def _attn_call(tf, s, b, wqkv, bqkv, wp, bp):
    return pl.pallas_call(
        _attn_body,
        grid=(NTOT // ABR,),
        in_specs=[pl.BlockSpec((ABR, D), lambda i: (i, 0))] + _ATTN_WSPECS,
        out_specs=pl.BlockSpec((ABR, D), lambda i: (i, 0)),
        out_shape=jax.ShapeDtypeStruct((NTOT, D), F32),
    )(tf, s, b, wqkv, bqkv, wp, bp)


def _attn_res_call(tf, y, s, b, wqkv, bqkv, wp, bp):
    return pl.pallas_call(
        _attn_res_body,
        grid=(NTOT // ABR,),
        in_specs=[pl.BlockSpec((ABR, D), lambda i: (i, 0)),
                  pl.BlockSpec((ABR, D), lambda i: (i, 0))] + _ATTN_WSPECS,
        out_specs=pl.BlockSpec((ABR, D), lambda i: (i, 0)),
        out_shape=jax.ShapeDtypeStruct((NTOT, D), F32),
    )(tf, y, s, b, wqkv, bqkv, wp, bp)


def _ffn_body(t_ref, s_ref, b_ref, w1_ref, b1_ref, w2_ref, b2_ref, o_ref):
    t = t_ref[...]
    h = _ln(t, s_ref[...], b_ref[...])
    a = jax.nn.gelu(_mm(h, w1_ref[...]) + b1_ref[...])
    o_ref[...] = t + _mm(a, w2_ref[...]) + b2_ref[...]


def _ffn_call(tf, s, b, w1, b1, w2, b2):
    return pl.pallas_call(
        _ffn_body,
        grid=(NTOT // ABR,),
        in_specs=[pl.BlockSpec((ABR, D), lambda i: (i, 0)),
                  pl.BlockSpec((1, D), lambda i: (0, 0)),
                  pl.BlockSpec((1, D), lambda i: (0, 0)),
                  pl.BlockSpec((D, FF), lambda i: (0, 0)),
                  pl.BlockSpec((1, FF), lambda i: (0, 0)),
                  pl.BlockSpec((FF, D), lambda i: (0, 0)),
                  pl.BlockSpec((1, D), lambda i: (0, 0))],
        out_specs=pl.BlockSpec((ABR, D), lambda i: (i, 0)),
        out_shape=jax.ShapeDtypeStruct((NTOT, D), F32),
    )(tf, s, b, w1, b1, w2, b2)


def _hlog_body(t_ref, s_ref, b_ref, re_ref, h_ref, lg_ref):
    h = _ln(t_ref[...], s_ref[...], b_ref[...])
    h_ref[...] = h
    lg_ref[...] = _cos_logits(h, re_ref[...])


def _hlog_call(tf, s, b, re):
    return pl.pallas_call(
        _hlog_body,
        grid=(NTOT // ABR,),
        in_specs=[pl.BlockSpec((ABR, D), lambda i: (i, 0)),
                  pl.BlockSpec((1, D), lambda i: (0, 0)),
                  pl.BlockSpec((1, D), lambda i: (0, 0)),
                  pl.BlockSpec((E, D), lambda i: (0, 0))],
        out_specs=[pl.BlockSpec((ABR, D), lambda i: (i, 0)),
                   pl.BlockSpec((ABR, E), lambda i: (i, 0))],
        out_shape=[jax.ShapeDtypeStruct((NTOT, D), F32),
                   jax.ShapeDtypeStruct((NTOT, E), F32)],
    )(tf, s, b, re)


def _route_body(lg_ref, dst_ref, bexp_ref):
    lg = lg_ref[...]
    amax = jnp.argmax(lg, axis=-1)
    oh = (jax.lax.broadcasted_iota(I32, (NTOT, E), 1)
          == amax[:, None]).astype(F32)
    tri = (jax.lax.broadcasted_iota(I32, (SCC, SCC), 0)
           > jax.lax.broadcasted_iota(I32, (SCC, SCC), 1)).astype(F32)
    offs = jnp.zeros((1, E), F32)
    parts = []
    for c in range(NTOT // SCC):
        ohc = oh[c * SCC:(c + 1) * SCC]
        excl = jax.lax.dot(tri, ohc, preferred_element_type=F32,
                           precision=jax.lax.Precision.HIGHEST)
        parts.append(excl + offs)
        offs = offs + ohc.sum(0, keepdims=True)
    rank = jnp.concatenate(parts, axis=0)
    counts = offs
    padded = jnp.ceil(counts / BLK) * BLK
    su = (jax.lax.broadcasted_iota(I32, (E, E), 0)
          < jax.lax.broadcasted_iota(I32, (E, E), 1)).astype(F32)
    base = jax.lax.dot(padded, su, preferred_element_type=F32,
                       precision=jax.lax.Precision.HIGHEST)
    dst = ((rank + base) * oh).sum(-1, keepdims=True)
    dst_ref[...] = dst.astype(I32).reshape(NW, NCH, CHUNK)
    jb = jax.lax.broadcasted_iota(I32, (NBLK, 1), 0).astype(F32) * BLK
    active = (jb >= base) & (jb < base + padded)
    eidx = jax.lax.broadcasted_iota(I32, (NBLK, E), 1).astype(F32)
    bexp_ref[...] = jnp.where(active, eidx, 0.).sum(-1, keepdims=True).astype(I32)


def _route_call(lg):
    return pl.pallas_call(
        _route_body,
        in_specs=[pl.BlockSpec((NTOT, E), lambda: (0, 0))],
        out_specs=[pl.BlockSpec((NW, NCH, CHUNK), lambda: (0, 0, 0)),
                   pl.BlockSpec((NBLK, 1), lambda: (0, 0))],
        out_shape=[jax.ShapeDtypeStruct((NW, NCH, CHUNK), I32),
                   jax.ShapeDtypeStruct((NBLK, 1), I32)],
    )(lg)


def _disp_body(h_hbm, idx_hbm, out_hbm, idx_v, rows_v, sem):
    wid = lax.axis_index("s") * 2 + lax.axis_index("c")
    pltpu.sync_copy(idx_hbm.at[wid], idx_v)
    for c in range(NCH):
        base = wid * (NCH * CHUNK) + c * CHUNK
        pltpu.sync_copy(h_hbm.at[pl.ds(base, CHUNK)], rows_v)
        pltpu.async_copy(rows_v, out_hbm.at[idx_v.at[c]], sem).wait()


def _comb_body(y_hbm, idx_hbm, out_hbm, idx_v, rows_v, sem):
    wid = lax.axis_index("s") * 2 + lax.axis_index("c")
    pltpu.sync_copy(idx_hbm.at[wid], idx_v)
    for c in range(NCH):
        base = wid * (NCH * CHUNK) + c * CHUNK
        pltpu.async_copy(y_hbm.at[idx_v.at[c]], rows_v, sem).wait()
        pltpu.sync_copy(rows_v, out_hbm.at[pl.ds(base, CHUNK)])


@functools.cache
def _sc_build():
    mesh = plsc.VectorSubcoreMesh(core_axis_name="c", subcore_axis_name="s",
                                  num_cores=2, num_subcores=16)
    scratch = [pltpu.VMEM((NCH, CHUNK), I32),
               pltpu.VMEM((CHUNK, D), F32),
               pltpu.SemaphoreType.DMA]
    disp = pl.kernel(_disp_body,
                     out_type=jax.ShapeDtypeStruct((CAP, D), F32),
                     mesh=mesh, scratch_types=scratch)
    comb = pl.kernel(_comb_body,
                     out_type=jax.ShapeDtypeStruct((NTOT, D), F32),
                     mesh=mesh, scratch_types=scratch)
    return disp, comb


def _sc_dispatch(h, idx):
    return _sc_build()[0](h, idx)


def _sc_combine(y, idx):
    return _sc_build()[1](y, idx)


def _gffn_body(bexp_ref, h_ref, re_ref, w1_ref, b1_ref, w2_ref, b2_ref, o_ref):
    e = bexp_ref[pl.program_id(0)]
    h = h_ref[...]
    lg = _cos_logits(h, re_ref[...])
    mx = lg.max(-1, keepdims=True)
    ex = jnp.exp(lg - mx)
    probs = ex / ex.sum(-1, keepdims=True)
    onehot_e = (jax.lax.broadcasted_iota(I32, (BLK, E), 1) == e).astype(F32)
    gate = (probs * onehot_e).sum(-1, keepdims=True)
    a = jax.nn.gelu(_mm(h, w1_ref[0]) + b1_ref[0])
    y = _mm(a, w2_ref[0]) + b2_ref[0]
    o_ref[...] = y * gate


def _gffn_call(bexp, sorted_h, re, w1, b1, w2, b2):
    grid_spec = pltpu.PrefetchScalarGridSpec(
        num_scalar_prefetch=1,
        grid=(NBLK,),
        in_specs=[pl.BlockSpec((BLK, D), lambda i, s: (i, 0)),
                  pl.BlockSpec((E, D), lambda i, s: (0, 0)),
                  pl.BlockSpec((1, D, FF), lambda i, s: (s[i], 0, 0)),
                  pl.BlockSpec((1, 1, FF), lambda i, s: (s[i], 0, 0)),
                  pl.BlockSpec((1, FF, D), lambda i, s: (s[i], 0, 0)),
                  pl.BlockSpec((1, 1, D), lambda i, s: (s[i], 0, 0))],
        out_specs=pl.BlockSpec((BLK, D), lambda i, s: (i, 0)),
    )
    return pl.pallas_call(
        _gffn_body,
        grid_spec=grid_spec,
        out_shape=jax.ShapeDtypeStruct((CAP, D), F32),
    )(bexp, sorted_h, re, w1, b1, w2, b2)


def _moe_sparse(tf, s, b, re, w1, b1, w2, b2):
    h, lg = _hlog_call(tf, s, b, re)
    idx, bexp = _route_call(lg)
    sorted_h = _sc_dispatch(h, idx)
    sorted_y = _gffn_call(bexp.reshape(NBLK), sorted_h, re,
                          _bf(w1), b1.reshape(E, 1, FF),
                          _bf(w2), b2.reshape(E, 1, D))
    return _sc_combine(sorted_y, idx)


def _head_body(t_ref, s_ref, b_ref, w_ref, bh_ref, o_ref):
    h = _ln(t_ref[...], s_ref[...], b_ref[...])
    o_ref[...] = _mm(h, w_ref[...]) + bh_ref[...]


def _head_call(tc, s, b, w, bh):
    B = tc.shape[0]
    return pl.pallas_call(
        _head_body,
        in_specs=[pl.BlockSpec((B, D), lambda: (0, 0)),
                  pl.BlockSpec((1, D), lambda: (0, 0)),
                  pl.BlockSpec((1, D), lambda: (0, 0)),
                  pl.BlockSpec((D, NC), lambda: (0, 0)),
                  pl.BlockSpec((1, NC), lambda: (0, 0))],
        out_specs=pl.BlockSpec((B, NC), lambda: (0, 0)),
        out_shape=jax.ShapeDtypeStruct((B, NC), F32),
    )(tc, s, b, w, bh)


def kernel(x, patch_w, patch_b, cls_tok, pos_emb, ln1_s, ln1_b, qkv_w, qkv_b,
           proj_w, proj_b, ln2_s, ln2_b, fc1_w, fc1_b, fc2_w, fc2_b,
           moe_w1, moe_b1, moe_w2, moe_b2, router_e, lnf_s, lnf_b,
           head_w, head_b):
    B = x.shape[0]
    p = x.reshape(B, 3, G, P, G, P).transpose(0, 2, 4, 3, 5, 1).reshape(B, G * G, 3 * P * P)
    t0 = _patch_call(p, _bf(patch_w), patch_b.reshape(1, D), pos_emb[:, 1:])
    cls_row = cls_tok + pos_emb[:, 0:1]
    t3 = jnp.concatenate([jnp.broadcast_to(cls_row, (B, 1, D)), t0,
                          jnp.zeros((B, TPI - NTOK, D), F32)], axis=1)
    tf = t3.reshape(NTOT, D)
    pending = None
    for i in range(DEPTH):
        args = (ln1_s[i].reshape(1, D), ln1_b[i].reshape(1, D),
                _bf(qkv_w[i]), qkv_b[i].reshape(1, 3 * D),
                _bf(proj_w[i]), proj_b[i].reshape(1, D))
        if pending is None:
            tf = _attn_call(tf, *args)
        else:
            tf = _attn_res_call(tf, pending, *args)
            pending = None
        if i in MOE_AT:
            j = MOE_AT[i]
            pending = _moe_sparse(tf, ln2_s[i].reshape(1, D),
                                  ln2_b[i].reshape(1, D),
                                  router_e[j], moe_w1[j], moe_b1[j],
                                  moe_w2[j], moe_b2[j])
        else:
            tf = _ffn_call(tf, ln2_s[i].reshape(1, D), ln2_b[i].reshape(1, D),
                           _bf(fc1_w[i]), fc1_b[i].reshape(1, FF),
                           _bf(fc2_w[i]), fc2_b[i].reshape(1, D))
    tc = tf.reshape(B, TPI, D)[:, 0]
    return _head_call(tc, lnf_s.reshape(1, D), lnf_b.reshape(1, D),
                      _bf(head_w), head_b.reshape(1, NC))
